# baseline (speedup 1.0000x reference)
import functools

import jax
import jax.numpy as jnp
from jax import lax
from jax.experimental import pallas as pl
from jax.experimental.pallas import tpu as pltpu
from jax.experimental.pallas import tpu_sc as plsc

Y_MIN = 0.0
Y_MAX = 1.0
N_BINS = 64
L = 16
NC = 2
NS = 16
NW = NC * NS
ROWS = 16384
COLS = 1024
RPW = ROWS // NW
CR = 16
NCH = RPW // CR
IV = CR * COLS // L
CV = COLS // L
U = 2
INV_STEP = float(N_BINS) / (Y_MAX + 1e-6 - Y_MIN)
EDGES_PAD = 72

_mesh = plsc.VectorSubcoreMesh(core_axis_name="c", subcore_axis_name="s")


@functools.partial(
    pl.kernel,
    mesh=_mesh,
    out_type=[
        jax.ShapeDtypeStruct((NW, N_BINS, L), jnp.float32),
        jax.ShapeDtypeStruct((NW, N_BINS, L), jnp.float32),
    ],
    scratch_types=[
        pltpu.VMEM((EDGES_PAD, L), jnp.float32),
        pltpu.VMEM((CR, COLS), jnp.float32),
        pltpu.VMEM((CR, COLS), jnp.float32),
        pltpu.VMEM((CR, COLS), jnp.float32),
        pltpu.VMEM((CR, COLS), jnp.float32),
        pltpu.VMEM((N_BINS, L), jnp.float32),
        pltpu.VMEM((N_BINS, L), jnp.float32),
        pltpu.SemaphoreType.DMA,
        pltpu.SemaphoreType.DMA,
        pltpu.SemaphoreType.DMA,
        pltpu.SemaphoreType.DMA,
    ],
    compiler_params=pltpu.CompilerParams(needs_layout_passes=False),
)
def _sc_binned(yp_hbm, yt_hbm, edges_hbm, cnt_out, sum_out,
               edges_v, ypb0, ytb0, ypb1, ytb1, cnt_v, sum_v,
               sp0, st0, sp1, st1):
    wid = lax.axis_index("s") * NC + lax.axis_index("c")
    base = wid * RPW
    pltpu.sync_copy(edges_hbm, edges_v)
    zeros = jnp.zeros((L,), jnp.float32)
    for b in range(N_BINS):
        cnt_v[b] = zeros
        sum_v[b] = zeros
    lane = lax.iota(jnp.int32, L)
    ones = jnp.ones((L,), jnp.float32)

    def copies(g, ypb, ytb, sp, st):
        off = base + lax.rem(g, NCH) * CR
        cp = pltpu.make_async_copy(yp_hbm.at[pl.ds(off, CR)], ypb, sp)
        ct = pltpu.make_async_copy(yt_hbm.at[pl.ds(off, CR)], ytb, st)
        return cp, ct

    def start(g, ypb, ytb, sp, st):
        cp, ct = copies(g, ypb, ytb, sp, st)
        cp.start()
        ct.start()

    def wait(g, ypb, ytb, sp, st):
        cp, ct = copies(g, ypb, ytb, sp, st)
        cp.wait()
        ct.wait()

    def process(ypb, ytb, row, col):
        yp = ypb[row, pl.ds(col, L)]
        yt = ytb[row, pl.ds(col, L)]
        d = yp - yt
        m = d * d
        v = m * m
        z = yt * INV_STEP + 8388608.0
        c = plsc.bitcast(z, jnp.int32) & 127
        e = plsc.load_gather(edges_v, [c, lane])
        bidx = jnp.where(e >= yt, c - 1, c)
        plsc.addupdate_scatter(cnt_v, [bidx, lane], ones)
        plsc.addupdate_scatter(sum_v, [bidx, lane], v)

    def compute(ypb, ytb):
        @plsc.parallel_loop(0, IV, unroll=U)
        def vec_body(i):
            row = lax.shift_right_logical(i, 6)
            col = lax.shift_left(lax.bitwise_and(i, CV - 1), 4)
            process(ypb, ytb, row, col)

    start(0, ypb0, ytb0, sp0, st0)

    def chunk_pair(h, carry):
        g = 2 * h
        start(g + 1, ypb1, ytb1, sp1, st1)
        wait(g, ypb0, ytb0, sp0, st0)
        compute(ypb0, ytb0)
        start(g + 2, ypb0, ytb0, sp0, st0)
        wait(g + 1, ypb1, ytb1, sp1, st1)
        compute(ypb1, ytb1)
        return carry

    lax.fori_loop(0, NCH // 2, chunk_pair, 0)
    wait(0, ypb0, ytb0, sp0, st0)
    pltpu.sync_copy(cnt_v, cnt_out.at[wid])
    pltpu.sync_copy(sum_v, sum_out.at[wid])


def _combine_body(cnt_ref, sum_ref, out_ref):
    cnt = jnp.sum(cnt_ref[...], axis=2)
    s = jnp.sum(sum_ref[...], axis=2)
    c_tot = jnp.sum(cnt, axis=0, keepdims=True)
    s_tot = jnp.sum(s, axis=0, keepdims=True)
    nonempty = c_tot > 0.0
    w = jnp.where(nonempty, 1.0 / (c_tot + 1e-6), 0.0)
    ws = jnp.sum(s_tot * w)
    ne = jnp.sum(jnp.where(nonempty, 1.0, 0.0))
    loss = jnp.where(ne == 0.0, 0.0, ws / ne)
    out_ref[...] = loss[None, None]


def kernel(y_pred, y_true):
    edges = jnp.linspace(Y_MIN, Y_MAX + 1e-6, N_BINS + 1, dtype=jnp.float32)
    edges = edges.at[0].set(-1.0)
    edges_pad = jnp.concatenate(
        [edges, jnp.full((EDGES_PAD - (N_BINS + 1),), 2.0, jnp.float32)])
    edges_rep = jnp.tile(edges_pad[:, None], (1, L))
    cnt2d, sum2d = _sc_binned(y_pred, y_true, edges_rep)
    loss2d = pl.pallas_call(
        _combine_body,
        out_shape=jax.ShapeDtypeStruct((1, 1), jnp.float32),
    )(cnt2d, sum2d)
    return loss2d[0, 0]


# speedup vs baseline: 1.0077x; 1.0077x over previous
import functools

import jax
import jax.numpy as jnp
from jax import lax
from jax.experimental import pallas as pl
from jax.experimental.pallas import tpu as pltpu
from jax.experimental.pallas import tpu_sc as plsc

Y_MIN = 0.0
Y_MAX = 1.0
N_BINS = 64
L = 16
NC = 2
NS = 16
NW = NC * NS
ROWS = 16384
COLS = 1024
RPW = ROWS // NW
CR = 8
NCH = RPW // CR
IV = CR * COLS // L
CV = COLS // L
U = 2
INV_STEP = float(N_BINS) / (Y_MAX + 1e-6 - Y_MIN)
EDGES_PAD = 72

_mesh = plsc.VectorSubcoreMesh(core_axis_name="c", subcore_axis_name="s")


@functools.partial(
    pl.kernel,
    mesh=_mesh,
    out_type=[
        jax.ShapeDtypeStruct((NW, N_BINS, L), jnp.float32),
        jax.ShapeDtypeStruct((NW, N_BINS, L), jnp.float32),
    ],
    scratch_types=[
        pltpu.VMEM((EDGES_PAD, L), jnp.float32),
        pltpu.VMEM((CR, COLS), jnp.float32),
        pltpu.VMEM((CR, COLS), jnp.float32),
        pltpu.VMEM((CR, COLS), jnp.float32),
        pltpu.VMEM((CR, COLS), jnp.float32),
        pltpu.VMEM((N_BINS, L), jnp.float32),
        pltpu.VMEM((N_BINS, L), jnp.float32),
        pltpu.SemaphoreType.DMA,
        pltpu.SemaphoreType.DMA,
        pltpu.SemaphoreType.DMA,
        pltpu.SemaphoreType.DMA,
    ],
    compiler_params=pltpu.CompilerParams(needs_layout_passes=False),
)
def _sc_binned(yp_hbm, yt_hbm, edges_hbm, cnt_out, sum_out,
               edges_v, ypb0, ytb0, ypb1, ytb1, cnt_v, sum_v,
               sp0, st0, sp1, st1):
    wid = lax.axis_index("s") * NC + lax.axis_index("c")
    base = wid * RPW
    pltpu.sync_copy(edges_hbm, edges_v)
    zeros = jnp.zeros((L,), jnp.float32)
    for b in range(N_BINS):
        cnt_v[b] = zeros
        sum_v[b] = zeros
    lane = lax.iota(jnp.int32, L)
    ones = jnp.ones((L,), jnp.float32)

    def copies(g, ypb, ytb, sp, st):
        off = base + lax.rem(g, NCH) * CR
        cp = pltpu.make_async_copy(yp_hbm.at[pl.ds(off, CR)], ypb, sp)
        ct = pltpu.make_async_copy(yt_hbm.at[pl.ds(off, CR)], ytb, st)
        return cp, ct

    def start(g, ypb, ytb, sp, st):
        cp, ct = copies(g, ypb, ytb, sp, st)
        cp.start()
        ct.start()

    def wait(g, ypb, ytb, sp, st):
        cp, ct = copies(g, ypb, ytb, sp, st)
        cp.wait()
        ct.wait()

    def process(ypb, ytb, row, col):
        yp = ypb[row, pl.ds(col, L)]
        yt = ytb[row, pl.ds(col, L)]
        d = yp - yt
        m = d * d
        v = m * m
        z = yt * INV_STEP + 8388608.0
        c = plsc.bitcast(z, jnp.int32) & 127
        e = plsc.load_gather(edges_v, [c, lane])
        bidx = jnp.where(e >= yt, c - 1, c)
        plsc.addupdate_scatter(cnt_v, [bidx, lane], ones)
        plsc.addupdate_scatter(sum_v, [bidx, lane], v)

    def compute(ypb, ytb):
        @plsc.parallel_loop(0, IV, unroll=U)
        def vec_body(i):
            row = lax.shift_right_logical(i, 6)
            col = lax.shift_left(lax.bitwise_and(i, CV - 1), 4)
            process(ypb, ytb, row, col)

    start(0, ypb0, ytb0, sp0, st0)

    def chunk_pair(h, carry):
        g = 2 * h
        start(g + 1, ypb1, ytb1, sp1, st1)
        wait(g, ypb0, ytb0, sp0, st0)
        compute(ypb0, ytb0)
        start(g + 2, ypb0, ytb0, sp0, st0)
        wait(g + 1, ypb1, ytb1, sp1, st1)
        compute(ypb1, ytb1)
        return carry

    lax.fori_loop(0, NCH // 2, chunk_pair, 0)
    wait(0, ypb0, ytb0, sp0, st0)
    pltpu.sync_copy(cnt_v, cnt_out.at[wid])
    pltpu.sync_copy(sum_v, sum_out.at[wid])


def _combine_body(cnt_ref, sum_ref, out_ref):
    cnt = jnp.sum(cnt_ref[...], axis=2)
    s = jnp.sum(sum_ref[...], axis=2)
    c_tot = jnp.sum(cnt, axis=0, keepdims=True)
    s_tot = jnp.sum(s, axis=0, keepdims=True)
    nonempty = c_tot > 0.0
    w = jnp.where(nonempty, 1.0 / (c_tot + 1e-6), 0.0)
    ws = jnp.sum(s_tot * w)
    ne = jnp.sum(jnp.where(nonempty, 1.0, 0.0))
    loss = jnp.where(ne == 0.0, 0.0, ws / ne)
    out_ref[...] = loss[None, None]


def kernel(y_pred, y_true):
    edges = jnp.linspace(Y_MIN, Y_MAX + 1e-6, N_BINS + 1, dtype=jnp.float32)
    edges = edges.at[0].set(-1.0)
    edges_pad = jnp.concatenate(
        [edges, jnp.full((EDGES_PAD - (N_BINS + 1),), 2.0, jnp.float32)])
    edges_rep = jnp.tile(edges_pad[:, None], (1, L))
    cnt2d, sum2d = _sc_binned(y_pred, y_true, edges_rep)
    loss2d = pl.pallas_call(
        _combine_body,
        out_shape=jax.ShapeDtypeStruct((1, 1), jnp.float32),
    )(cnt2d, sum2d)
    return loss2d[0, 0]
